# Initial kernel scaffold; baseline (speedup 1.0000x reference)
import jax
import jax.numpy as jnp
from jax.experimental import pallas as pl


def kernel(x, edge_index, batch, seq, global_f, params):
    raise NotImplementedError("write your pallas kernel here")


# speedup vs baseline: 6.3782x; 6.3782x over previous
import functools

import jax
import jax.numpy as jnp
from jax import lax
from jax.experimental import pallas as pl
from jax.experimental.pallas import tpu as pltpu
from jax.experimental.pallas import tpu_sc as plsc

N_NODES = 10000
N_EDGES = 320000
D = 128
G = 64
HEADS = 4
NLAYER = 3

NC = 2
NS = 16
NW = NC * NS
E_PER_W = N_EDGES // NW
CHUNK = 80
NCHUNK = E_PER_W // CHUNK
NPAD = ((N_NODES + 127) // 128) * 128
ROWS_T = NPAD // NS

BLK = 1000
NB = N_NODES // BLK


def _leaky(v):
    return jnp.where(v > 0, v, 0.1 * v)


def _sc_edge_agg(h_hbm, srcs_hbm, dsts_hbm, zeros_hbm, out_hbm,
                 src_v, dst_v, rows_v, agg_sh, sem):
    c = lax.axis_index("c")
    s = lax.axis_index("s")
    wid = c * NS + s
    pltpu.sync_copy(srcs_hbm.at[wid], src_v)
    pltpu.sync_copy(dsts_hbm.at[wid], dst_v)
    pltpu.sync_copy(zeros_hbm, agg_sh.at[pl.ds(s * ROWS_T, ROWS_T)])
    plsc.subcore_barrier()

    def body(j, carry):
        pltpu.async_copy(h_hbm.at[src_v.at[j]], rows_v, sem).wait()
        pltpu.sync_copy(rows_v, agg_sh.at[dst_v.at[j]], add=True)
        return carry

    lax.fori_loop(0, NCHUNK, body, 0)
    plsc.subcore_barrier()
    pltpu.sync_copy(agg_sh.at[pl.ds(s * ROWS_T, ROWS_T)],
                    out_hbm.at[c, pl.ds(s * ROWS_T, ROWS_T)])


@functools.lru_cache(maxsize=1)
def _get_sc_edge_agg_call():
    return functools.partial(
        pl.kernel,
        mesh=plsc.VectorSubcoreMesh(core_axis_name="c", subcore_axis_name="s"),
        out_type=jax.ShapeDtypeStruct((NC, NPAD, D), jnp.float32),
        scratch_types=[
            pltpu.VMEM((NCHUNK, CHUNK), jnp.int32),
            pltpu.VMEM((NCHUNK, CHUNK), jnp.int32),
            pltpu.VMEM((CHUNK, D), jnp.float32),
            pltpu.VMEM_SHARED((NPAD, D), jnp.float32),
            pltpu.SemaphoreType.DMA,
        ],
    )(_sc_edge_agg)


def _tc_layer(p0_ref, p1_ref, h_ref, wrel_ref, wroot_ref, b_ref, o_ref):
    agg = p0_ref[...] + p1_ref[...]
    o_ref[...] = (
        jnp.dot(agg, wrel_ref[...], preferred_element_type=jnp.float32)
        + jnp.dot(h_ref[...], wroot_ref[...], preferred_element_type=jnp.float32)
        + b_ref[...]
    )


def _layer_update(p0, p1, h, wrel, wroot, brel):
    row = pl.BlockSpec((BLK, D), lambda i: (i, 0))
    full = pl.BlockSpec((D, D), lambda i: (0, 0))
    bias = pl.BlockSpec((1, D), lambda i: (0, 0))
    return pl.pallas_call(
        _tc_layer,
        grid=(NB,),
        in_specs=[row, row, row, full, full, bias],
        out_specs=row,
        out_shape=jax.ShapeDtypeStruct((N_NODES, D), jnp.float32),
    )(p0, p1, h, wrel, wroot, brel.reshape(1, D))


def _tc_epilogue(h_ref, batch_ref, seq_ref, gf_ref, ws_ref, bs_ref, wg_ref,
                 bg_ref, wq_ref, bq_ref, wk_ref, bk_ref, wv_ref, bv_ref,
                 wp_ref, bp_ref, scale_ref, w1_ref, b1_ref, w2_ref, b2_ref,
                 o_ref, pool_ref):
    j = pl.program_id(0)

    @pl.when(j == 0)
    def _():
        pool_ref[...] = jnp.zeros_like(pool_ref)

    b = batch_ref[0, 0, :]
    gids = lax.broadcasted_iota(jnp.int32, (G, BLK), 0)
    mask = (b[None, :] == gids).astype(jnp.float32)
    pool_ref[...] += jnp.dot(mask, h_ref[...], preferred_element_type=jnp.float32)

    @pl.when(j == NB - 1)
    def _():
        graph_rep = pool_ref[...]
        seq_rep = _leaky(
            jnp.dot(seq_ref[...], ws_ref[...], preferred_element_type=jnp.float32)
            + bs_ref[...])
        glob = _leaky(
            jnp.dot(gf_ref[...], wg_ref[...], preferred_element_type=jnp.float32)
            + bg_ref[...])
        toks = (graph_rep, seq_rep, glob)
        inv_scale = 1.0 / scale_ref[0, 0]
        qf = [jnp.dot(t, wq_ref[...], preferred_element_type=jnp.float32)
              + bq_ref[...] for t in toks]
        kf = [jnp.dot(t, wk_ref[...], preferred_element_type=jnp.float32)
              + bk_ref[...] for t in toks]
        vf = [jnp.dot(t, wv_ref[...], preferred_element_type=jnp.float32)
              + bv_ref[...] for t in toks]
        osum_heads = []
        for hh in range(HEADS):
            sl = slice(hh * D, (hh + 1) * D)
            q = [t[:, sl] for t in qf]
            k = [t[:, sl] for t in kf]
            v = [t[:, sl] for t in vf]
            o_h = jnp.zeros((G, D), jnp.float32)
            for i in range(3):
                sc = [jnp.sum(q[i] * k[jj], axis=1, keepdims=True) * inv_scale
                      for jj in range(3)]
                m = jnp.maximum(jnp.maximum(sc[0], sc[1]), sc[2])
                e = [jnp.exp(x - m) for x in sc]
                denom = e[0] + e[1] + e[2]
                att = [x / denom for x in e]
                o_h = o_h + att[0] * v[0] + att[1] * v[1] + att[2] * v[2]
            osum_heads.append(o_h)
        osum = jnp.concatenate(osum_heads, axis=1)
        a1 = (jnp.dot(osum, wp_ref[...], preferred_element_type=jnp.float32)
              + 3.0 * bp_ref[...])
        h1 = _leaky(
            jnp.dot(a1, w1_ref[...], preferred_element_type=jnp.float32)
            + b1_ref[...])
        o_ref[...] = (
            jnp.dot(h1, w2_ref[...], preferred_element_type=jnp.float32)
            + b2_ref[...])


def _epilogue(h, batch3, seq, global_f, p):
    def fixed(a):
        return a, pl.BlockSpec(a.shape, lambda i: tuple(0 for _ in a.shape))

    ops = [
        (h, pl.BlockSpec((BLK, D), lambda i: (i, 0))),
        (batch3, pl.BlockSpec((1, 1, BLK), lambda i: (i, 0, 0))),
        fixed(seq),
        fixed(global_f),
        fixed(p['Ws']), fixed(p['bs'].reshape(1, D)),
        fixed(p['Wg']), fixed(p['bg'].reshape(1, D)),
        fixed(p['Wq']), fixed(p['bq'].reshape(1, HEADS * D)),
        fixed(p['Wk']), fixed(p['bk'].reshape(1, HEADS * D)),
        fixed(p['Wv']), fixed(p['bv'].reshape(1, HEADS * D)),
        fixed(p['Wp']), fixed(p['bp'].reshape(1, D)),
        fixed(p['scale'].reshape(1, 1)),
        fixed(p['W1']), fixed(p['b1'].reshape(1, D)),
        fixed(p['W2']), fixed(p['b2'].reshape(1, 1)),
    ]
    args = [a for a, _ in ops]
    specs = [s for _, s in ops]
    return pl.pallas_call(
        _tc_epilogue,
        grid=(NB,),
        in_specs=specs,
        out_specs=pl.BlockSpec((G, 1), lambda i: (0, 0)),
        out_shape=jax.ShapeDtypeStruct((G, 1), jnp.float32),
        scratch_shapes=[pltpu.VMEM((G, D), jnp.float32)],
    )(*args)


def kernel(x, edge_index, batch, seq, global_f, params):
    src = edge_index[0]
    dst = edge_index[1]
    srcs_r = src.reshape(NW, NCHUNK, CHUNK)
    dsts_r = dst.reshape(NW, NCHUNK, CHUNK)
    zeros = jnp.zeros((ROWS_T, D), jnp.float32)
    batch3 = batch.reshape(NB, 1, BLK)

    h = x
    for l in range(NLAYER):
        part = _get_sc_edge_agg_call()(h, srcs_r, dsts_r, zeros)
        p0 = part[0, :N_NODES]
        p1 = part[1, :N_NODES]
        h = _layer_update(p0, p1, h, params['Wrel%d' % l],
                          params['Wroot%d' % l], params['brel%d' % l])
    return _epilogue(h, batch3, seq, global_f, params)
